# baseline (speedup 1.0000x reference)
import functools

import jax
import jax.numpy as jnp
from jax import lax
from jax.experimental import pallas as pl
from jax.experimental.pallas import tpu as pltpu
from jax.experimental.pallas import tpu_sc as plsc

F = 26
V = 100000
D = 16
B = 16384

NC = 2
NS = 16
NW = NC * NS

ROWS = F * D
PER_W = ROWS // NW

SEG = 33408
OFFS = (0, SEG, 2 * SEG)
SIZES = (SEG, SEG, 33152)
VT = 99968
TW = V - VT
NSEG = 3

OCH = 4096
NOCH = B // OCH


def _body(tab_hbm, tail_hbm, idx_hbm, out_hbm, buf0, buf1, ob, idx_v,
          tail_v, wb0, wb1, semA, semB, tsem, wsem0, wsem1):
    wid = lax.axis_index("s") * NC + lax.axis_index("c")
    r0 = wid * PER_W

    bufs = (buf0, buf1)
    sems = (semA, semB)
    wbufs = (wb0, wb1)
    wsems = (wsem0, wsem1)
    pend = [None, None]
    wpend = [None, None]

    def issue(g):
        j, k = divmod(g, NSEG)
        b = g % 2
        pend[b] = pltpu.async_copy(
            tab_hbm.at[r0 + j, pl.ds(OFFS[k], SIZES[k])],
            bufs[b].at[pl.ds(0, SIZES[k])],
            sems[b],
        )

    issue(0)
    tpend = [
        pltpu.async_copy(tail_hbm.at[r0 + jj], tail_v.at[jj], tsem)
        for jj in range(PER_W)
    ]

    for j in range(PER_W):
        r = r0 + j
        f = r // D

        if j == 0:
            pltpu.sync_copy(idx_hbm.at[f], idx_v)
        else:
            f_prev = (r - 1) // D

            @pl.when(f != f_prev)
            def _():
                pltpu.sync_copy(idx_hbm.at[f], idx_v)

        for k in range(NSEG):
            g = NSEG * j + k
            buf = bufs[g % 2]
            pend[g % 2].wait()
            if g + 1 < NSEG * PER_W:
                issue(g + 1)

            if k == 0:
                def p0(gi):
                    iv = idx_v[pl.ds(gi * 16, 16)]
                    loc = jnp.minimum(iv, SIZES[0] - 1)
                    ob[pl.ds(gi * 16, 16)] = plsc.load_gather(buf, [loc])

                pl.loop(0, B // 16, unroll=8)(p0)
            elif k == 1:
                def p1(gi):
                    iv = idx_v[pl.ds(gi * 16, 16)]
                    loc = jnp.clip(iv - OFFS[1], 0, SIZES[1] - 1)
                    gv = plsc.load_gather(buf, [loc])
                    prev = ob[pl.ds(gi * 16, 16)]
                    ob[pl.ds(gi * 16, 16)] = jnp.where(iv >= OFFS[1], gv, prev)

                pl.loop(0, B // 16, unroll=8)(p1)
            else:
                if tpend is not None:
                    for tp in tpend:
                        tp.wait()
                    tpend = None
                jrow = jnp.full((16,), j, jnp.int32)
                for c in range(NOCH):
                    w = c % 2
                    if wpend[w] is not None:
                        wpend[w].wait()
                    wb = wbufs[w]

                    def p2(gi, c=c, wb=wb, jrow=jrow):
                        iv = idx_v[pl.ds(c * OCH + gi * 16, 16)]
                        loc = jnp.clip(iv - OFFS[2], 0, SIZES[2] - 1)
                        gv = plsc.load_gather(buf, [loc])
                        loct = jnp.clip(iv - VT, 0, TW - 1)
                        gt = plsc.load_gather(tail_v, [jrow, loct])
                        prev = ob[pl.ds(c * OCH + gi * 16, 16)]
                        v = jnp.where(iv >= OFFS[2], gv, prev)
                        wb[pl.ds(gi * 16, 16)] = jnp.where(iv >= VT, gt, v)

                    pl.loop(0, OCH // 16, unroll=8)(p2)
                    wpend[w] = pltpu.async_copy(
                        wb, out_hbm.at[r, pl.ds(c * OCH, OCH)], wsems[w]
                    )

    for p in wpend:
        if p is not None:
            p.wait()


@jax.jit
def _run(tab_t, tail_t, idx_t):
    kern = functools.partial(
        pl.kernel,
        mesh=plsc.VectorSubcoreMesh(core_axis_name="c", subcore_axis_name="s"),
        out_type=jax.ShapeDtypeStruct((ROWS, B), jnp.float32),
        scratch_types=[
            pltpu.VMEM((SEG,), jnp.float32),
            pltpu.VMEM((SEG,), jnp.float32),
            pltpu.VMEM((B,), jnp.float32),
            pltpu.VMEM((B,), jnp.int32),
            pltpu.VMEM((PER_W, TW), jnp.float32),
            pltpu.VMEM((OCH,), jnp.float32),
            pltpu.VMEM((OCH,), jnp.float32),
            pltpu.SemaphoreType.DMA,
            pltpu.SemaphoreType.DMA,
            pltpu.SemaphoreType.DMA,
            pltpu.SemaphoreType.DMA,
            pltpu.SemaphoreType.DMA,
        ],
        compiler_params=pltpu.CompilerParams(
            use_tc_tiling_on_sc=True, needs_layout_passes=False
        ),
    )(_body)
    return kern(tab_t, tail_t, idx_t)


def kernel(indices, tables):
    tab_t = tables.transpose(0, 2, 1).reshape(ROWS, V)
    tail_t = tables[:, VT:, :].transpose(0, 2, 1).reshape(ROWS, TW)
    idx_t = indices.T.astype(jnp.int32)
    out_t = _run(tab_t, tail_t, idx_t)
    return out_t.T


# speedup vs baseline: 4.7149x; 4.7149x over previous
import functools

import jax
import jax.numpy as jnp
from jax import lax
from jax.experimental import pallas as pl
from jax.experimental.pallas import tpu as pltpu
from jax.experimental.pallas import tpu_sc as plsc

F = 26
V = 100000
D = 16
B = 16384

NC = 2
NS = 16
NW = NC * NS

ROWS = F * D
PER_W = ROWS // NW
OCH = 4096
NOCH = B // OCH


def _body(tab_hbm, idx_hbm, out_hbm, row_v, idx_v, ob0, ob1, sem0, sem1):
    wid = lax.axis_index("s") * NC + lax.axis_index("c")
    r0 = wid * PER_W

    obufs = (ob0, ob1)
    sems = (sem0, sem1)
    pending = [None, None]

    for j in range(PER_W):
        r = r0 + j
        f = r // D

        if j == 0:
            pltpu.sync_copy(idx_hbm.at[f], idx_v)
        else:
            f_prev = (r - 1) // D

            @pl.when(f != f_prev)
            def _():
                pltpu.sync_copy(idx_hbm.at[f], idx_v)

        pltpu.sync_copy(tab_hbm.at[r], row_v)

        for c in range(NOCH):
            k = c % 2
            if pending[k] is not None:
                pending[k].wait()
            ob = obufs[k]

            @plsc.parallel_loop(0, OCH // 16, unroll=8)
            def gather(g, c=c, ob=ob):
                iv = idx_v[pl.ds(c * OCH + g * 16, 16)]
                ob[pl.ds(g * 16, 16)] = plsc.load_gather(row_v, [iv])

            pending[k] = pltpu.async_copy(
                ob, out_hbm.at[r, pl.ds(c * OCH, OCH)], sems[k]
            )

    for p in pending:
        if p is not None:
            p.wait()


@jax.jit
def _run(tab_t, idx_t):
    kern = functools.partial(
        pl.kernel,
        mesh=plsc.VectorSubcoreMesh(core_axis_name="c", subcore_axis_name="s"),
        out_type=jax.ShapeDtypeStruct((ROWS, B), jnp.float32),
        scratch_types=[
            pltpu.VMEM((V,), jnp.float32),
            pltpu.VMEM((B,), jnp.int32),
            pltpu.VMEM((OCH,), jnp.float32),
            pltpu.VMEM((OCH,), jnp.float32),
            pltpu.SemaphoreType.DMA,
            pltpu.SemaphoreType.DMA,
        ],
        compiler_params=pltpu.CompilerParams(
            use_tc_tiling_on_sc=True, needs_layout_passes=False
        ),
    )(_body)
    return kern(tab_t, idx_t)


def kernel(indices, tables):
    tab_t = tables.transpose(0, 2, 1).reshape(ROWS, V)
    idx_t = indices.T.astype(jnp.int32)
    out_t = _run(tab_t, idx_t)
    return out_t.T
